# baseline (speedup 1.0000x reference)
import functools

import jax
import jax.numpy as jnp
from jax import lax
from jax.experimental import pallas as pl
from jax.experimental.pallas import tpu as pltpu
from jax.experimental.pallas import tpu_sc as plsc

NC, NS = 2, 16
NW = NC * NS
N = 10000
E = 320000
D = 128
NG = 64
CH = 128
NCHUNK = 80
EPW = CH * NCHUNK
EPAD = NW * EPW
TOTCH = EPAD // CH
C0 = 40
C1 = 120
NPAD = 10240
RPS = NPAD // NS
ZB = 128


def _mesh():
    return plsc.VectorSubcoreMesh(
        core_axis_name="c", subcore_axis_name="s",
        num_cores=NC, num_subcores=NS)


@functools.cache
def _sc_degree_kernel():
    return pl.kernel(
        _sc_degree_body,
        out_type=jax.ShapeDtypeStruct((NC, NPAD, D), jnp.float32),
        mesh=_mesh(),
        scratch_types=[
            pltpu.VMEM((CH,), jnp.int32),
            pltpu.VMEM((CH, D), jnp.float32),
            pltpu.VMEM_SHARED((NPAD, D), jnp.float32),
        ],
    )


def _sc_degree(dstz):
    return _sc_degree_kernel()(dstz)


def _sc_degree_body(dstz_hbm, out_hbm, idx_d, buf, acc_sh):
    c = lax.axis_index("c")
    s = lax.axis_index("s")
    wid = s * NC + c

    def _fill(val):
        def _f(k, _):
            buf[k // 8, pl.ds((k % 8) * 16, 16)] = jnp.full((16,), val,
                                                            jnp.float32)
            return 0
        return _f

    lax.fori_loop(0, CH * (D // 16), _fill(0.0), 0)

    def _z(k, _):
        pltpu.sync_copy(buf, acc_sh.at[pl.ds(s * RPS + k * CH, CH)])
        return 0
    lax.fori_loop(0, RPS // CH, _z, 0)
    plsc.subcore_barrier()

    lax.fori_loop(0, CH * (D // 16), _fill(1.0), 0)

    def _chunk(j, _):
        pltpu.sync_copy(dstz_hbm.at[wid * NCHUNK + j], idx_d)
        pltpu.sync_copy(buf, acc_sh.at[idx_d], add=True)
        return 0
    lax.fori_loop(0, NCHUNK, _chunk, 0)
    plsc.subcore_barrier()

    pltpu.sync_copy(acc_sh.at[pl.ds(s * RPS, RPS)],
                    out_hbm.at[c, pl.ds(s * RPS, RPS)])


@functools.cache
def _sc_scatter_kernel():
    return pl.kernel(
        _sc_scatter_body,
        out_type=jax.ShapeDtypeStruct((NC, NPAD, D), jnp.float32),
        mesh=_mesh(),
        scratch_types=[
            pltpu.VMEM((CH,), jnp.int32),
            pltpu.VMEM((CH,), jnp.int32),
            pltpu.VMEM((CH,), jnp.int32),
            pltpu.VMEM((CH,), jnp.int32),
            pltpu.VMEM((CH, D), jnp.float32),
            pltpu.VMEM((CH, D), jnp.float32),
            pltpu.VMEM_SHARED((NPAD, D), jnp.float32),
            pltpu.SemaphoreType.DMA,
            pltpu.SemaphoreType.DMA,
        ],
    )


def _sc_scatter(g, srcz, dstz):
    return _sc_scatter_kernel()(g, srcz, dstz)


def _sc_scatter_body(g_hbm, srcz_hbm, dstz_hbm, out_hbm, idx_s0, idx_d0,
                     idx_s1, idx_d1, rows0, rows1, acc_sh, sem0, sem1):
    c = lax.axis_index("c")
    s = lax.axis_index("s")
    start = jnp.where(c == 0, s * C0, NS * C0 + s * C1)
    cnt = jnp.where(c == 0, C0, C1)
    idx_s = (idx_s0, idx_s1)
    idx_d = (idx_d0, idx_d1)
    rows = (rows0, rows1)
    sems = (sem0, sem1)

    def _zb(k, _):
        rows0[k // 8, pl.ds((k % 8) * 16, 16)] = jnp.zeros((16,), jnp.float32)
        return 0
    lax.fori_loop(0, CH * (D // 16), _zb, 0)

    def _z(k, _):
        pltpu.sync_copy(rows0, acc_sh.at[pl.ds(s * RPS + k * CH, CH)])
        return 0
    lax.fori_loop(0, RPS // CH, _z, 0)
    plsc.subcore_barrier()

    for b in range(2):
        pltpu.sync_copy(srcz_hbm.at[start + b], idx_s[b])
        pltpu.sync_copy(dstz_hbm.at[start + b], idx_d[b])
        pltpu.async_copy(g_hbm.at[idx_s[b]], rows[b], sems[b])

    def _grp(gi, _):
        for b in range(2):
            j = gi * 2 + b
            pltpu.make_async_copy(g_hbm.at[idx_s[b]], rows[b], sems[b]).wait()
            pltpu.sync_copy(rows[b], acc_sh.at[idx_d[b]], add=True)
            pltpu.sync_copy(srcz_hbm.at[start + j + 2], idx_s[b])
            pltpu.sync_copy(dstz_hbm.at[start + j + 2], idx_d[b])
            pltpu.async_copy(g_hbm.at[idx_s[b]], rows[b], sems[b])
        return 0
    lax.fori_loop(0, cnt // 2 - 1, _grp, 0)

    for b in range(2):
        pltpu.make_async_copy(g_hbm.at[idx_s[b]], rows[b], sems[b]).wait()
        pltpu.sync_copy(rows[b], acc_sh.at[idx_d[b]], add=True)
    plsc.subcore_barrier()

    pltpu.sync_copy(acc_sh.at[pl.ds(s * RPS, RPS)],
                    out_hbm.at[c, pl.ds(s * RPS, RPS)])


_BLK = 1000
_GRID = N // _BLK


def _tc_prep_body(degp_ref, x_ref, w_ref, dinv_ref, g_ref):
    deg = degp_ref[0, :, :16] + degp_ref[1, :, :16] + 1.0
    dinv = lax.rsqrt(deg)
    dinv_ref[...] = dinv
    hw = jnp.dot(x_ref[...], w_ref[...], preferred_element_type=jnp.float32)
    g_ref[...] = hw * dinv[:, :1]


def _tc_prep(degp, x, w1):
    return pl.pallas_call(
        _tc_prep_body,
        grid=(_GRID,),
        in_specs=[
            pl.BlockSpec((NC, _BLK, D), lambda i: (0, i, 0)),
            pl.BlockSpec((_BLK, D), lambda i: (i, 0)),
            pl.BlockSpec((D, D), lambda i: (0, 0)),
        ],
        out_specs=[
            pl.BlockSpec((_BLK, 16), lambda i: (i, 0)),
            pl.BlockSpec((_BLK, D), lambda i: (i, 0)),
        ],
        out_shape=[
            jax.ShapeDtypeStruct((N, 16), jnp.float32),
            jax.ShapeDtypeStruct((N, D), jnp.float32),
        ],
    )(degp, x, w1)


def _tc_mid_body(p_ref, g_ref, dinv_ref, b_ref, w_ref, gn_ref):
    dv = dinv_ref[:, :1]
    pre = (p_ref[0] + p_ref[1] + g_ref[...]) * dv + b_ref[...]
    h = jnp.maximum(pre, 0.0)
    gn_ref[...] = jnp.dot(h, w_ref[...],
                          preferred_element_type=jnp.float32) * dv


def _tc_mid(p, g, dinv, b, w):
    return pl.pallas_call(
        _tc_mid_body,
        grid=(_GRID,),
        in_specs=[
            pl.BlockSpec((NC, _BLK, D), lambda i: (0, i, 0)),
            pl.BlockSpec((_BLK, D), lambda i: (i, 0)),
            pl.BlockSpec((_BLK, 16), lambda i: (i, 0)),
            pl.BlockSpec((1, D), lambda i: (0, 0)),
            pl.BlockSpec((D, D), lambda i: (0, 0)),
        ],
        out_specs=pl.BlockSpec((_BLK, D), lambda i: (i, 0)),
        out_shape=jax.ShapeDtypeStruct((N, D), jnp.float32),
    )(p, g, dinv, b.reshape(1, D), w)


def _tc_final_body(p_ref, g_ref, dinv_ref, b_ref, batch_ref, wl_ref, bl_ref,
                   out_ref, sums_scr, cnt_scr):
    i = pl.program_id(0)

    @pl.when(i == 0)
    def _():
        sums_scr[...] = jnp.zeros_like(sums_scr)
        cnt_scr[...] = jnp.zeros_like(cnt_scr)

    dv = dinv_ref[:, :1]
    h5 = (p_ref[0] + p_ref[1] + g_ref[...]) * dv + b_ref[...]
    gid = batch_ref[:, :1]
    oneh = (gid == lax.broadcasted_iota(jnp.int32, (1, NG), 1))
    oneh = oneh.astype(jnp.float32)
    dn = (((0,), (0,)), ((), ()))
    sums_scr[...] += lax.dot_general(oneh, h5, dn,
                                     preferred_element_type=jnp.float32)
    cnt_scr[...] += lax.dot_general(oneh, jnp.ones_like(h5), dn,
                                    preferred_element_type=jnp.float32)

    @pl.when(i == pl.num_programs(0) - 1)
    def _():
        pooled = sums_scr[...] / jnp.maximum(cnt_scr[...], 1.0)
        out_ref[...] = jnp.dot(pooled, wl_ref[...],
                               preferred_element_type=jnp.float32) + bl_ref[...]


def _tc_final(p, g, dinv, b5, batch16, w_lin, b_lin):
    ncls = w_lin.shape[1]
    return pl.pallas_call(
        _tc_final_body,
        grid=(_GRID,),
        in_specs=[
            pl.BlockSpec((NC, _BLK, D), lambda i: (0, i, 0)),
            pl.BlockSpec((_BLK, D), lambda i: (i, 0)),
            pl.BlockSpec((_BLK, 16), lambda i: (i, 0)),
            pl.BlockSpec((1, D), lambda i: (0, 0)),
            pl.BlockSpec((_BLK, 16), lambda i: (i, 0)),
            pl.BlockSpec((D, ncls), lambda i: (0, 0)),
            pl.BlockSpec((1, ncls), lambda i: (0, 0)),
        ],
        out_specs=pl.BlockSpec((NG, ncls), lambda i: (0, 0)),
        out_shape=jax.ShapeDtypeStruct((NG, ncls), jnp.float32),
        scratch_shapes=[
            pltpu.VMEM((NG, D), jnp.float32),
            pltpu.VMEM((NG, D), jnp.float32),
        ],
    )(p, g, dinv, b5.reshape(1, D), batch16, w_lin, b_lin.reshape(1, ncls))


@jax.jit
def kernel(x, edge_index, batch, W1, b1, W2, b2, W3, b3, W4, b4, W5, b5,
           W_lin, b_lin):
    src = edge_index[0].astype(jnp.int32)
    dst = edge_index[1].astype(jnp.int32)
    npd = EPAD - E
    pad_src = jnp.zeros((npd,), jnp.int32)
    pad_dst = N + (jnp.arange(npd, dtype=jnp.int32) % (NPAD - N))
    srcz = jnp.concatenate([src, pad_src]).reshape(TOTCH, CH)
    dstz = jnp.concatenate([dst, pad_dst]).reshape(TOTCH, CH)
    batch16 = jnp.broadcast_to(batch.astype(jnp.int32)[:, None], (N, 16))

    degp = _sc_degree(dstz)
    dinv, g = _tc_prep(degp, x, W1)
    for b, w in ((b1, W2), (b2, W3), (b3, W4), (b4, W5)):
        p = _sc_scatter(g, srcz, dstz)
        g = _tc_mid(p, g, dinv, b, w)
    p = _sc_scatter(g, srcz, dstz)
    return _tc_final(p, g, dinv, b5, batch16, W_lin, b_lin)


# speedup vs baseline: 1.1755x; 1.1755x over previous
import functools

import jax
import jax.numpy as jnp
from jax import lax
from jax.experimental import pallas as pl
from jax.experimental.pallas import tpu as pltpu
from jax.experimental.pallas import tpu_sc as plsc

NC, NS = 2, 16
NW = NC * NS
N = 10000
E = 320000
D = 128
NG = 64
CH = 128
NCHUNK = 80
EPW = CH * NCHUNK
EPAD = NW * EPW
TOTCH = EPAD // CH
C0 = 120
C1 = 40
NPAD = 10240
RPS = NPAD // NS
ZB = 128


def _mesh():
    return plsc.VectorSubcoreMesh(
        core_axis_name="c", subcore_axis_name="s",
        num_cores=NC, num_subcores=NS)


@functools.cache
def _sc_degree_kernel():
    return pl.kernel(
        _sc_degree_body,
        out_type=jax.ShapeDtypeStruct((NC, NPAD, D), jnp.float32),
        mesh=_mesh(),
        scratch_types=[
            pltpu.VMEM((CH,), jnp.int32),
            pltpu.VMEM((CH, D), jnp.float32),
            pltpu.VMEM_SHARED((NPAD, D), jnp.float32),
        ],
    )


def _sc_degree(dstz):
    return _sc_degree_kernel()(dstz)


def _sc_degree_body(dstz_hbm, out_hbm, idx_d, buf, acc_sh):
    c = lax.axis_index("c")
    s = lax.axis_index("s")
    wid = s * NC + c

    def _fill(val):
        def _f(k, _):
            buf[k // 8, pl.ds((k % 8) * 16, 16)] = jnp.full((16,), val,
                                                            jnp.float32)
            return 0
        return _f

    lax.fori_loop(0, CH * (D // 16), _fill(0.0), 0)

    def _z(k, _):
        pltpu.sync_copy(buf, acc_sh.at[pl.ds(s * RPS + k * CH, CH)])
        return 0
    lax.fori_loop(0, RPS // CH, _z, 0)
    plsc.subcore_barrier()

    lax.fori_loop(0, CH * (D // 16), _fill(1.0), 0)

    def _chunk(j, _):
        pltpu.sync_copy(dstz_hbm.at[wid * NCHUNK + j], idx_d)
        pltpu.sync_copy(buf, acc_sh.at[idx_d], add=True)
        return 0
    lax.fori_loop(0, NCHUNK, _chunk, 0)
    plsc.subcore_barrier()

    pltpu.sync_copy(acc_sh.at[pl.ds(s * RPS, RPS)],
                    out_hbm.at[c, pl.ds(s * RPS, RPS)])


@functools.cache
def _sc_scatter_kernel():
    return pl.kernel(
        _sc_scatter_body,
        out_type=jax.ShapeDtypeStruct((NC, NPAD, D), jnp.float32),
        mesh=_mesh(),
        scratch_types=[
            pltpu.VMEM((CH,), jnp.int32),
            pltpu.VMEM((CH,), jnp.int32),
            pltpu.VMEM((CH,), jnp.int32),
            pltpu.VMEM((CH,), jnp.int32),
            pltpu.VMEM((CH, D), jnp.float32),
            pltpu.VMEM((CH, D), jnp.float32),
            pltpu.VMEM_SHARED((NPAD, D), jnp.float32),
            pltpu.SemaphoreType.DMA,
            pltpu.SemaphoreType.DMA,
        ],
    )


def _sc_scatter(g, srcz, dstz):
    return _sc_scatter_kernel()(g, srcz, dstz)


def _sc_scatter_body(g_hbm, srcz_hbm, dstz_hbm, out_hbm, idx_s0, idx_d0,
                     idx_s1, idx_d1, rows0, rows1, acc_sh, sem0, sem1):
    c = lax.axis_index("c")
    s = lax.axis_index("s")
    start = jnp.where(c == 0, s * C0, NS * C0 + s * C1)
    cnt = jnp.where(c == 0, C0, C1)
    idx_s = (idx_s0, idx_s1)
    idx_d = (idx_d0, idx_d1)
    rows = (rows0, rows1)
    sems = (sem0, sem1)

    def _zb(k, _):
        rows0[k // 8, pl.ds((k % 8) * 16, 16)] = jnp.zeros((16,), jnp.float32)
        return 0
    lax.fori_loop(0, CH * (D // 16), _zb, 0)

    def _z(k, _):
        pltpu.sync_copy(rows0, acc_sh.at[pl.ds(s * RPS + k * CH, CH)])
        return 0
    lax.fori_loop(0, RPS // CH, _z, 0)
    plsc.subcore_barrier()

    for b in range(2):
        pltpu.sync_copy(srcz_hbm.at[start + b], idx_s[b])
        pltpu.sync_copy(dstz_hbm.at[start + b], idx_d[b])
        pltpu.async_copy(g_hbm.at[idx_s[b]], rows[b], sems[b])

    def _grp(gi, _):
        for b in range(2):
            j = gi * 2 + b
            pltpu.make_async_copy(g_hbm.at[idx_s[b]], rows[b], sems[b]).wait()
            pltpu.sync_copy(rows[b], acc_sh.at[idx_d[b]], add=True)
            pltpu.sync_copy(srcz_hbm.at[start + j + 2], idx_s[b])
            pltpu.sync_copy(dstz_hbm.at[start + j + 2], idx_d[b])
            pltpu.async_copy(g_hbm.at[idx_s[b]], rows[b], sems[b])
        return 0
    lax.fori_loop(0, cnt // 2 - 1, _grp, 0)

    for b in range(2):
        pltpu.make_async_copy(g_hbm.at[idx_s[b]], rows[b], sems[b]).wait()
        pltpu.sync_copy(rows[b], acc_sh.at[idx_d[b]], add=True)
    plsc.subcore_barrier()

    pltpu.sync_copy(acc_sh.at[pl.ds(s * RPS, RPS)],
                    out_hbm.at[c, pl.ds(s * RPS, RPS)])


_BLK = 1000
_GRID = N // _BLK


def _tc_prep_body(degp_ref, x_ref, w_ref, dinv_ref, g_ref):
    deg = degp_ref[0, :, :16] + degp_ref[1, :, :16] + 1.0
    dinv = lax.rsqrt(deg)
    dinv_ref[...] = dinv
    hw = jnp.dot(x_ref[...], w_ref[...], preferred_element_type=jnp.float32)
    g_ref[...] = hw * dinv[:, :1]


def _tc_prep(degp, x, w1):
    return pl.pallas_call(
        _tc_prep_body,
        grid=(_GRID,),
        in_specs=[
            pl.BlockSpec((NC, _BLK, D), lambda i: (0, i, 0)),
            pl.BlockSpec((_BLK, D), lambda i: (i, 0)),
            pl.BlockSpec((D, D), lambda i: (0, 0)),
        ],
        out_specs=[
            pl.BlockSpec((_BLK, 16), lambda i: (i, 0)),
            pl.BlockSpec((_BLK, D), lambda i: (i, 0)),
        ],
        out_shape=[
            jax.ShapeDtypeStruct((N, 16), jnp.float32),
            jax.ShapeDtypeStruct((N, D), jnp.float32),
        ],
    )(degp, x, w1)


def _tc_mid_body(p_ref, g_ref, dinv_ref, b_ref, w_ref, gn_ref):
    dv = dinv_ref[:, :1]
    pre = (p_ref[0] + p_ref[1] + g_ref[...]) * dv + b_ref[...]
    h = jnp.maximum(pre, 0.0)
    gn_ref[...] = jnp.dot(h, w_ref[...],
                          preferred_element_type=jnp.float32) * dv


def _tc_mid(p, g, dinv, b, w):
    return pl.pallas_call(
        _tc_mid_body,
        grid=(_GRID,),
        in_specs=[
            pl.BlockSpec((NC, _BLK, D), lambda i: (0, i, 0)),
            pl.BlockSpec((_BLK, D), lambda i: (i, 0)),
            pl.BlockSpec((_BLK, 16), lambda i: (i, 0)),
            pl.BlockSpec((1, D), lambda i: (0, 0)),
            pl.BlockSpec((D, D), lambda i: (0, 0)),
        ],
        out_specs=pl.BlockSpec((_BLK, D), lambda i: (i, 0)),
        out_shape=jax.ShapeDtypeStruct((N, D), jnp.float32),
    )(p, g, dinv, b.reshape(1, D), w)


def _tc_final_body(p_ref, g_ref, dinv_ref, b_ref, batch_ref, wl_ref, bl_ref,
                   out_ref, sums_scr, cnt_scr):
    i = pl.program_id(0)

    @pl.when(i == 0)
    def _():
        sums_scr[...] = jnp.zeros_like(sums_scr)
        cnt_scr[...] = jnp.zeros_like(cnt_scr)

    dv = dinv_ref[:, :1]
    h5 = (p_ref[0] + p_ref[1] + g_ref[...]) * dv + b_ref[...]
    gid = batch_ref[:, :1]
    oneh = (gid == lax.broadcasted_iota(jnp.int32, (1, NG), 1))
    oneh = oneh.astype(jnp.float32)
    dn = (((0,), (0,)), ((), ()))
    sums_scr[...] += lax.dot_general(oneh, h5, dn,
                                     preferred_element_type=jnp.float32)
    cnt_scr[...] += lax.dot_general(oneh, jnp.ones_like(h5), dn,
                                    preferred_element_type=jnp.float32)

    @pl.when(i == pl.num_programs(0) - 1)
    def _():
        pooled = sums_scr[...] / jnp.maximum(cnt_scr[...], 1.0)
        out_ref[...] = jnp.dot(pooled, wl_ref[...],
                               preferred_element_type=jnp.float32) + bl_ref[...]


def _tc_final(p, g, dinv, b5, batch16, w_lin, b_lin):
    ncls = w_lin.shape[1]
    return pl.pallas_call(
        _tc_final_body,
        grid=(_GRID,),
        in_specs=[
            pl.BlockSpec((NC, _BLK, D), lambda i: (0, i, 0)),
            pl.BlockSpec((_BLK, D), lambda i: (i, 0)),
            pl.BlockSpec((_BLK, 16), lambda i: (i, 0)),
            pl.BlockSpec((1, D), lambda i: (0, 0)),
            pl.BlockSpec((_BLK, 16), lambda i: (i, 0)),
            pl.BlockSpec((D, ncls), lambda i: (0, 0)),
            pl.BlockSpec((1, ncls), lambda i: (0, 0)),
        ],
        out_specs=pl.BlockSpec((NG, ncls), lambda i: (0, 0)),
        out_shape=jax.ShapeDtypeStruct((NG, ncls), jnp.float32),
        scratch_shapes=[
            pltpu.VMEM((NG, D), jnp.float32),
            pltpu.VMEM((NG, D), jnp.float32),
        ],
    )(p, g, dinv, b5.reshape(1, D), batch16, w_lin, b_lin.reshape(1, ncls))


@jax.jit
def kernel(x, edge_index, batch, W1, b1, W2, b2, W3, b3, W4, b4, W5, b5,
           W_lin, b_lin):
    src = edge_index[0].astype(jnp.int32)
    dst = edge_index[1].astype(jnp.int32)
    npd = EPAD - E
    pad_src = jnp.zeros((npd,), jnp.int32)
    pad_dst = N + (jnp.arange(npd, dtype=jnp.int32) % (NPAD - N))
    srcz = jnp.concatenate([src, pad_src]).reshape(TOTCH, CH)
    dstz = jnp.concatenate([dst, pad_dst]).reshape(TOTCH, CH)
    batch16 = jnp.broadcast_to(batch.astype(jnp.int32)[:, None], (N, 16))

    degp = _sc_degree(dstz)
    dinv, g = _tc_prep(degp, x, W1)
    for b, w in ((b1, W2), (b2, W3), (b3, W4), (b4, W5)):
        p = _sc_scatter(g, srcz, dstz)
        g = _tc_mid(p, g, dinv, b, w)
    p = _sc_scatter(g, srcz, dstz)
    return _tc_final(p, g, dinv, b5, batch16, W_lin, b_lin)
